# baseline (speedup 1.0000x reference)
import functools

import jax
import jax.numpy as jnp
from jax import lax
from jax.experimental import pallas as pl
from jax.experimental.pallas import tpu as pltpu
from jax.experimental.pallas import tpu_sc as plsc

N = 10000
E = 320000
D = 128
P = 64
DEPTH = 3

NC = 2
NS = 16
NW = NC * NS

K = 80
SUBS = 42
SCN = 3
EPT = SCN * SUBS * K
EPAD = NW * EPT

NPAD = 10240
RPT = NPAD // NS
ZR = 128
DPT = NPAD // NS

_mesh = plsc.VectorSubcoreMesh(core_axis_name="c", subcore_axis_name="s")


@functools.partial(
    pl.kernel,
    out_type=jax.ShapeDtypeStruct((NC, NPAD), jnp.float32),
    mesh=_mesh,
    scratch_types=[
        pltpu.VMEM((SUBS, K), jnp.int32),
        pltpu.VMEM((SUBS, K), jnp.float32),
        pltpu.VMEM((DPT,), jnp.float32),
        pltpu.VMEM_SHARED((NPAD,), jnp.float32),
        pltpu.SemaphoreType.DMA,
    ],
)
def _sc_deg(pdst_hbm, pew_hbm, out_hbm, dbuf, ewb, zbuf, acc, ssem):
    cid = lax.axis_index("c")
    sid = lax.axis_index("s")
    wid = cid * NS + sid

    def zrow(i, carry):
        zbuf[pl.ds(i * 16, 16)] = jnp.zeros((16,), jnp.float32)
        return carry

    lax.fori_loop(0, DPT // 16, zrow, 0)
    pltpu.sync_copy(zbuf, acc.at[pl.ds(sid * DPT, DPT)])
    plsc.subcore_barrier()

    def superchunk(s, carry):
        pltpu.sync_copy(pdst_hbm.at[wid, s], dbuf)
        pltpu.sync_copy(pew_hbm.at[wid, s], ewb)
        for s8 in range(SUBS):
            pltpu.async_copy(ewb.at[s8], acc.at[dbuf.at[s8]], ssem,
                             add=True)
        for s8 in range(SUBS):
            pltpu.make_async_copy(ewb.at[s8], acc.at[dbuf.at[s8]],
                                  ssem).wait()
        return carry

    lax.fori_loop(0, SCN, superchunk, 0)
    plsc.subcore_barrier()
    pltpu.sync_copy(acc.at[pl.ds(sid * DPT, DPT)],
                    out_hbm.at[cid, pl.ds(sid * DPT, DPT)])


@functools.partial(
    pl.kernel,
    out_type=jax.ShapeDtypeStruct((NC, NPAD, D), jnp.float32),
    mesh=_mesh,
    scratch_types=[
        pltpu.VMEM((SUBS, K), jnp.int32),
        pltpu.VMEM((SUBS, K), jnp.int32),
        pltpu.VMEM((SUBS, K), jnp.float32),
        pltpu.VMEM((K, D), jnp.float32),
        pltpu.VMEM((K, D), jnp.float32),
        pltpu.VMEM((K, D), jnp.float32),
        pltpu.VMEM_SHARED((NPAD, D), jnp.float32),
        pltpu.SemaphoreType.DMA,
        pltpu.SemaphoreType.DMA,
        pltpu.SemaphoreType.DMA,
        pltpu.SemaphoreType.DMA,
        pltpu.SemaphoreType.DMA,
        pltpu.SemaphoreType.DMA,
        pltpu.SemaphoreType.DMA,
    ],
)
def _sc_agg(g_hbm, psrc_hbm, pdst_hbm, pew_hbm, out_hbm,
            sbuf, dbuf, ewb, rows0, rows1, rows2, acc,
            gsem0, gsem1, gsem2, ssem0, ssem1, ssem2, isem):
    cid = lax.axis_index("c")
    sid = lax.axis_index("s")
    wid = cid * NS + sid

    rows = (rows0, rows1, rows2)
    gsem = (gsem0, gsem1, gsem2)
    ssem = (ssem0, ssem1, ssem2)

    def zrow(i, carry):
        z = jnp.zeros((16,), jnp.float32)
        for j in range(D // 16):
            rows0[i, pl.ds(j * 16, 16)] = z
        return carry

    lax.fori_loop(0, K, zrow, 0)
    for z in range(RPT // K):
        pltpu.sync_copy(rows0, acc.at[pl.ds(sid * RPT + z * K, K)])
    rem = RPT - (RPT // K) * K
    if rem:
        pltpu.sync_copy(rows0.at[pl.ds(0, rem)],
                        acc.at[pl.ds(sid * RPT + (RPT // K) * K, rem)])
    del rem
    plsc.subcore_barrier()

    def _swait(r):
        pltpu.make_async_copy(rows[r], acc.at[dbuf.at[0]], ssem[r]).wait()

    def _gwait(r):
        pltpu.make_async_copy(g_hbm.at[sbuf.at[0]], rows[r], gsem[r]).wait()

    def _scale(r, c):
        def sgrp(g, c2, _r=r, _c=c):
            ev = ewb[_c, pl.ds(g * 16, 16)]
            for k in range(16):
                sv = jnp.full((16,), ev[k], jnp.float32)
                rr = g * 16 + k
                for j in range(D // 16):
                    rows[_r][rr, pl.ds(j * 16, 16)] = (
                        rows[_r][rr, pl.ds(j * 16, 16)] * sv)
            return c2

        lax.fori_loop(0, K // 16, sgrp, 0)

    pltpu.sync_copy(pdst_hbm.at[wid, 0], dbuf)
    pltpu.async_copy(rows0, acc.at[dbuf.at[0]], ssem0, add=True)
    pltpu.async_copy(rows0, acc.at[dbuf.at[0]], ssem1, add=True)
    pltpu.async_copy(rows0, acc.at[dbuf.at[0]], ssem2, add=True)
    pltpu.async_copy(psrc_hbm.at[wid, 0], sbuf, isem)
    pltpu.async_copy(pew_hbm.at[wid, 0], ewb, isem)

    NR = SUBS // 3

    def superchunk(s, carry):
        pltpu.make_async_copy(psrc_hbm.at[wid, 0], sbuf, isem).wait()
        pltpu.make_async_copy(pew_hbm.at[wid, 0], ewb, isem).wait()
        _swait(0)
        _swait(1)
        _swait(2)
        pltpu.sync_copy(pdst_hbm.at[wid, s], dbuf)
        pltpu.async_copy(g_hbm.at[sbuf.at[0]], rows0, gsem0)
        pltpu.async_copy(g_hbm.at[sbuf.at[1]], rows1, gsem1)

        def rnd(p, c2):
            c0 = 3 * p

            @pl.when(p > 0)
            def _():
                _swait(2)
            pltpu.async_copy(g_hbm.at[sbuf.at[c0 + 2]], rows2, gsem2)
            _gwait(0)
            _scale(0, c0)
            pltpu.async_copy(rows0, acc.at[dbuf.at[c0]], ssem0, add=True)

            @pl.when(p < NR - 1)
            def _():
                _swait(0)
                pltpu.async_copy(g_hbm.at[sbuf.at[c0 + 3]], rows0, gsem0)
            _gwait(1)
            _scale(1, c0 + 1)
            pltpu.async_copy(rows1, acc.at[dbuf.at[c0 + 1]], ssem1, add=True)

            @pl.when(p < NR - 1)
            def _():
                _swait(1)
                pltpu.async_copy(g_hbm.at[sbuf.at[c0 + 4]], rows1, gsem1)
            _gwait(2)
            _scale(2, c0 + 2)
            pltpu.async_copy(rows2, acc.at[dbuf.at[c0 + 2]], ssem2, add=True)
            return c2

        lax.fori_loop(0, NR, rnd, 0)
        sn = lax.rem(s + 1, SCN)
        pltpu.async_copy(psrc_hbm.at[wid, sn], sbuf, isem)
        pltpu.async_copy(pew_hbm.at[wid, sn], ewb, isem)
        return carry

    lax.fori_loop(0, SCN, superchunk, 0)
    pltpu.make_async_copy(psrc_hbm.at[wid, 0], sbuf, isem).wait()
    pltpu.make_async_copy(pew_hbm.at[wid, 0], ewb, isem).wait()
    _swait(0)
    _swait(1)
    _swait(2)
    plsc.subcore_barrier()
    pltpu.sync_copy(acc.at[pl.ds(sid * RPT, RPT)],
                    out_hbm.at[cid, pl.ds(sid * RPT, RPT)])


@functools.partial(
    pl.kernel,
    out_type=jax.ShapeDtypeStruct((P, D), jnp.float32),
    mesh=_mesh,
    scratch_types=[
        pltpu.VMEM((P,), jnp.int32),
        pltpu.VMEM((P,), jnp.int32),
        pltpu.VMEM((P, D), jnp.float32),
        pltpu.VMEM((P, D), jnp.float32),
        pltpu.VMEM((P, D), jnp.float32),
        pltpu.VMEM((P,), jnp.float32),
        pltpu.VMEM((D,), jnp.float32),
        pltpu.SemaphoreType.DMA,
    ],
)
def _sc_tail(p3_hbm, g3_hbm, dinv_hbm, b3_hbm, pos_hbm, out_hbm,
             posv, pidx, ra, rb, rc, dv, bv, sem):
    cid = lax.axis_index("c")
    sid = lax.axis_index("s")

    @pl.when(jnp.logical_and(cid == 0, sid == 0))
    def _():
        pltpu.sync_copy(pos_hbm, posv)
        pltpu.sync_copy(b3_hbm, bv)
        for t in range(P // 16):
            v = posv[pl.ds(t * 16, 16)]
            v = jnp.minimum(jnp.maximum(v, 0), N - 1)
            pidx[pl.ds(t * 16, 16)] = v
        pltpu.async_copy(p3_hbm.at[0].at[pidx], ra, sem)
        pltpu.async_copy(p3_hbm.at[1].at[pidx], rb, sem)
        pltpu.async_copy(g3_hbm.at[pidx], rc, sem)
        pltpu.async_copy(dinv_hbm.at[pidx], dv, sem)
        pltpu.make_async_copy(p3_hbm.at[0].at[pidx], ra, sem).wait()
        pltpu.make_async_copy(p3_hbm.at[1].at[pidx], rb, sem).wait()
        pltpu.make_async_copy(g3_hbm.at[pidx], rc, sem).wait()
        pltpu.make_async_copy(dinv_hbm.at[pidx], dv, sem).wait()

        def crow(t, carry):
            ev = dv[pl.ds(t * 16, 16)]
            for k in range(16):
                sv = jnp.full((16,), ev[k], jnp.float32)
                rr = t * 16 + k
                for j in range(D // 16):
                    cs = pl.ds(j * 16, 16)
                    ra[rr, cs] = (ra[rr, cs] + rb[rr, cs] + rc[rr, cs]) * sv \
                        + bv[cs]
            return carry

        lax.fori_loop(0, P // 16, crow, 0)
        pltpu.sync_copy(ra, out_hbm)


BR = 1000
G = N // BR


def _tc_mm_body(x_ref, w_ref, hw_ref):
    hw_ref[...] = jnp.dot(x_ref[...], w_ref[...],
                          preferred_element_type=jnp.float32)


def _tc_mm(x, w):
    return pl.pallas_call(
        _tc_mm_body,
        grid=(G,),
        in_specs=[
            pl.BlockSpec((BR, D), lambda i: (i, 0)),
            pl.BlockSpec((D, D), lambda i: (0, 0)),
        ],
        out_specs=pl.BlockSpec((BR, D), lambda i: (i, 0)),
        out_shape=jax.ShapeDtypeStruct((N, D), jnp.float32),
    )(x, w)


def _tc_pre_body(degp_ref, hw_ref, g_ref, dinv_ref):
    deg = degp_ref[0] + degp_ref[1] + 1.0
    dinv = lax.rsqrt(deg)
    g_ref[...] = dinv * hw_ref[...]
    dinv_ref[...] = dinv


def _tc_pre(degp, hw):
    return pl.pallas_call(
        _tc_pre_body,
        grid=(G,),
        in_specs=[
            pl.BlockSpec((NC, BR, 1), lambda i: (0, i, 0)),
            pl.BlockSpec((BR, D), lambda i: (i, 0)),
        ],
        out_specs=[
            pl.BlockSpec((BR, D), lambda i: (i, 0)),
            pl.BlockSpec((BR, 1), lambda i: (i, 0)),
        ],
        out_shape=[
            jax.ShapeDtypeStruct((N, D), jnp.float32),
            jax.ShapeDtypeStruct((N, 1), jnp.float32),
        ],
    )(degp, hw)


def _tc_mid_body(parts_ref, g_ref, dinv_ref, b_ref, w_ref, gn_ref):
    s = parts_ref[0] + parts_ref[1] + g_ref[...]
    emb = dinv_ref[...] * s + b_ref[...]
    h = jnp.tanh(emb)
    hw = jnp.dot(h, w_ref[...], preferred_element_type=jnp.float32)
    gn_ref[...] = dinv_ref[...] * hw


def _tc_mid(parts, g, dinv, b, w):
    return pl.pallas_call(
        _tc_mid_body,
        grid=(G,),
        in_specs=[
            pl.BlockSpec((NC, BR, D), lambda i: (0, i, 0)),
            pl.BlockSpec((BR, D), lambda i: (i, 0)),
            pl.BlockSpec((BR, 1), lambda i: (i, 0)),
            pl.BlockSpec((1, D), lambda i: (0, 0)),
            pl.BlockSpec((D, D), lambda i: (0, 0)),
        ],
        out_specs=pl.BlockSpec((BR, D), lambda i: (i, 0)),
        out_shape=jax.ShapeDtypeStruct((N, D), jnp.float32),
    )(parts, g, dinv, b, w)


def kernel(x, edge_index, edge_weight, pos, W1, b1, W2, b2, W3, b3):
    src = edge_index[0]
    dst = edge_index[1]

    pad = EPAD - E
    fillr = (jnp.arange(pad, dtype=jnp.int32) * 131) % N
    psrc = jnp.concatenate([src, fillr]).reshape(NW, SCN, SUBS, K)
    pdst = jnp.concatenate([dst, fillr]).reshape(NW, SCN, SUBS, K)
    pew = jnp.concatenate(
        [edge_weight, jnp.zeros((pad,), jnp.float32)]).reshape(
            NW, SCN, SUBS, K)

    hw1 = _tc_mm(x, W1)
    degp = _sc_deg(pdst, pew)
    degp3 = degp.reshape(NC, NPAD, 1)

    g1, dinv = _tc_pre(degp3, hw1)

    p1 = _sc_agg(g1, psrc, pdst, pew)
    g2 = _tc_mid(p1, g1, dinv, b1.reshape(1, D), W2)
    p2 = _sc_agg(g2, psrc, pdst, pew)
    g3 = _tc_mid(p2, g2, dinv, b2.reshape(1, D), W3)
    p3 = _sc_agg(g3, psrc, pdst, pew)

    rows = _sc_tail(p3, g3, dinv.reshape(N), b3, pos)
    fill = jnp.full((P, D), -float(DEPTH), dtype=jnp.float32)
    embs = jnp.where((pos == -1)[:, None], fill, rows)
    return embs[None]


# speedup vs baseline: 1.0044x; 1.0044x over previous
import functools

import jax
import jax.numpy as jnp
from jax import lax
from jax.experimental import pallas as pl
from jax.experimental.pallas import tpu as pltpu
from jax.experimental.pallas import tpu_sc as plsc

N = 10000
E = 320000
D = 128
P = 64
DEPTH = 3

NC = 2
NS = 16
NW = NC * NS

K = 80
SUBS = 42
SCN = 3
EPT = SCN * SUBS * K
EPAD = NW * EPT

NPAD = 10240
RPT = NPAD // NS
ZR = 128
DPT = NPAD // NS

_mesh = plsc.VectorSubcoreMesh(core_axis_name="c", subcore_axis_name="s")


@functools.partial(
    pl.kernel,
    out_type=jax.ShapeDtypeStruct((NC, NPAD), jnp.float32),
    mesh=_mesh,
    scratch_types=[
        pltpu.VMEM((SUBS, K), jnp.int32),
        pltpu.VMEM((SUBS, K), jnp.float32),
        pltpu.VMEM((DPT,), jnp.float32),
        pltpu.VMEM_SHARED((NPAD,), jnp.float32),
        pltpu.SemaphoreType.DMA,
    ],
)
def _sc_deg(pdst_hbm, pew_hbm, out_hbm, dbuf, ewb, zbuf, acc, ssem):
    cid = lax.axis_index("c")
    sid = lax.axis_index("s")
    wid = cid * NS + sid

    def zrow(i, carry):
        zbuf[pl.ds(i * 16, 16)] = jnp.zeros((16,), jnp.float32)
        return carry

    lax.fori_loop(0, DPT // 16, zrow, 0)
    pltpu.sync_copy(zbuf, acc.at[pl.ds(sid * DPT, DPT)])
    plsc.subcore_barrier()

    def superchunk(s, carry):
        pltpu.sync_copy(pdst_hbm.at[wid, s], dbuf)
        pltpu.sync_copy(pew_hbm.at[wid, s], ewb)
        for s8 in range(SUBS):
            pltpu.async_copy(ewb.at[s8], acc.at[dbuf.at[s8]], ssem,
                             add=True)
        for s8 in range(SUBS):
            pltpu.make_async_copy(ewb.at[s8], acc.at[dbuf.at[s8]],
                                  ssem).wait()
        return carry

    lax.fori_loop(0, SCN, superchunk, 0)
    plsc.subcore_barrier()
    pltpu.sync_copy(acc.at[pl.ds(sid * DPT, DPT)],
                    out_hbm.at[cid, pl.ds(sid * DPT, DPT)])


@functools.partial(
    pl.kernel,
    out_type=jax.ShapeDtypeStruct((NC, NPAD, D), jnp.float32),
    mesh=_mesh,
    scratch_types=[
        pltpu.VMEM((SUBS, K), jnp.int32),
        pltpu.VMEM((SUBS, K), jnp.int32),
        pltpu.VMEM((SUBS, K), jnp.float32),
        pltpu.VMEM((K, D), jnp.float32),
        pltpu.VMEM((K, D), jnp.float32),
        pltpu.VMEM((K, D), jnp.float32),
        pltpu.VMEM_SHARED((NPAD, D), jnp.float32),
        pltpu.SemaphoreType.DMA,
        pltpu.SemaphoreType.DMA,
        pltpu.SemaphoreType.DMA,
        pltpu.SemaphoreType.DMA,
        pltpu.SemaphoreType.DMA,
        pltpu.SemaphoreType.DMA,
        pltpu.SemaphoreType.DMA,
    ],
)
def _sc_agg(g_hbm, psrc_hbm, pdst_hbm, pew_hbm, out_hbm,
            sbuf, dbuf, ewb, rows0, rows1, rows2, acc,
            gsem0, gsem1, gsem2, ssem0, ssem1, ssem2, isem):
    cid = lax.axis_index("c")
    sid = lax.axis_index("s")
    wid = cid * NS + sid

    rows = (rows0, rows1, rows2)
    gsem = (gsem0, gsem1, gsem2)
    ssem = (ssem0, ssem1, ssem2)

    def zrow(i, carry):
        z = jnp.zeros((16,), jnp.float32)
        for j in range(D // 16):
            rows0[i, pl.ds(j * 16, 16)] = z
        return carry

    lax.fori_loop(0, K, zrow, 0)
    for z in range(RPT // K):
        pltpu.async_copy(rows0, acc.at[pl.ds(sid * RPT + z * K, K)], isem)
    rem = RPT - (RPT // K) * K
    if rem:
        pltpu.sync_copy(rows0.at[pl.ds(0, rem)],
                        acc.at[pl.ds(sid * RPT + (RPT // K) * K, rem)])
    del rem
    for z in range(RPT // K):
        pltpu.make_async_copy(rows0, acc.at[pl.ds(sid * RPT + z * K, K)],
                              isem).wait()
    plsc.subcore_barrier()

    def _swait(r):
        pltpu.make_async_copy(rows[r], acc.at[dbuf.at[0]], ssem[r]).wait()

    def _gwait(r):
        pltpu.make_async_copy(g_hbm.at[sbuf.at[0]], rows[r], gsem[r]).wait()

    def _scale(r, c):
        def sgrp(g, c2, _r=r, _c=c):
            ev = ewb[_c, pl.ds(g * 16, 16)]
            for k in range(16):
                sv = jnp.full((16,), ev[k], jnp.float32)
                rr = g * 16 + k
                for j in range(D // 16):
                    rows[_r][rr, pl.ds(j * 16, 16)] = (
                        rows[_r][rr, pl.ds(j * 16, 16)] * sv)
            return c2

        lax.fori_loop(0, K // 16, sgrp, 0)

    pltpu.sync_copy(pdst_hbm.at[wid, 0], dbuf)
    pltpu.async_copy(rows0, acc.at[dbuf.at[0]], ssem0, add=True)
    pltpu.async_copy(rows0, acc.at[dbuf.at[0]], ssem1, add=True)
    pltpu.async_copy(rows0, acc.at[dbuf.at[0]], ssem2, add=True)
    pltpu.async_copy(psrc_hbm.at[wid, 0], sbuf, isem)
    pltpu.async_copy(pew_hbm.at[wid, 0], ewb, isem)

    NR = SUBS // 3

    def superchunk(s, carry):
        pltpu.make_async_copy(psrc_hbm.at[wid, 0], sbuf, isem).wait()
        pltpu.make_async_copy(pew_hbm.at[wid, 0], ewb, isem).wait()
        _swait(0)
        _swait(1)
        _swait(2)
        pltpu.sync_copy(pdst_hbm.at[wid, s], dbuf)
        pltpu.async_copy(g_hbm.at[sbuf.at[0]], rows0, gsem0)
        pltpu.async_copy(g_hbm.at[sbuf.at[1]], rows1, gsem1)

        def rnd(p, c2):
            c0 = 3 * p

            @pl.when(p > 0)
            def _():
                _swait(2)
            pltpu.async_copy(g_hbm.at[sbuf.at[c0 + 2]], rows2, gsem2)
            _gwait(0)
            _scale(0, c0)
            pltpu.async_copy(rows0, acc.at[dbuf.at[c0]], ssem0, add=True)

            @pl.when(p < NR - 1)
            def _():
                _swait(0)
                pltpu.async_copy(g_hbm.at[sbuf.at[c0 + 3]], rows0, gsem0)
            _gwait(1)
            _scale(1, c0 + 1)
            pltpu.async_copy(rows1, acc.at[dbuf.at[c0 + 1]], ssem1, add=True)

            @pl.when(p < NR - 1)
            def _():
                _swait(1)
                pltpu.async_copy(g_hbm.at[sbuf.at[c0 + 4]], rows1, gsem1)
            _gwait(2)
            _scale(2, c0 + 2)
            pltpu.async_copy(rows2, acc.at[dbuf.at[c0 + 2]], ssem2, add=True)
            return c2

        lax.fori_loop(0, NR, rnd, 0)
        sn = lax.rem(s + 1, SCN)
        pltpu.async_copy(psrc_hbm.at[wid, sn], sbuf, isem)
        pltpu.async_copy(pew_hbm.at[wid, sn], ewb, isem)
        return carry

    lax.fori_loop(0, SCN, superchunk, 0)
    pltpu.make_async_copy(psrc_hbm.at[wid, 0], sbuf, isem).wait()
    pltpu.make_async_copy(pew_hbm.at[wid, 0], ewb, isem).wait()
    _swait(0)
    _swait(1)
    _swait(2)
    plsc.subcore_barrier()
    pltpu.sync_copy(acc.at[pl.ds(sid * RPT, RPT)],
                    out_hbm.at[cid, pl.ds(sid * RPT, RPT)])


@functools.partial(
    pl.kernel,
    out_type=jax.ShapeDtypeStruct((P, D), jnp.float32),
    mesh=_mesh,
    scratch_types=[
        pltpu.VMEM((P,), jnp.int32),
        pltpu.VMEM((P,), jnp.int32),
        pltpu.VMEM((P, D), jnp.float32),
        pltpu.VMEM((P, D), jnp.float32),
        pltpu.VMEM((P, D), jnp.float32),
        pltpu.VMEM((P,), jnp.float32),
        pltpu.VMEM((D,), jnp.float32),
        pltpu.SemaphoreType.DMA,
    ],
)
def _sc_tail(p3_hbm, g3_hbm, dinv_hbm, b3_hbm, pos_hbm, out_hbm,
             posv, pidx, ra, rb, rc, dv, bv, sem):
    cid = lax.axis_index("c")
    sid = lax.axis_index("s")

    @pl.when(jnp.logical_and(cid == 0, sid == 0))
    def _():
        pltpu.sync_copy(pos_hbm, posv)
        pltpu.sync_copy(b3_hbm, bv)
        for t in range(P // 16):
            v = posv[pl.ds(t * 16, 16)]
            v = jnp.minimum(jnp.maximum(v, 0), N - 1)
            pidx[pl.ds(t * 16, 16)] = v
        pltpu.async_copy(p3_hbm.at[0].at[pidx], ra, sem)
        pltpu.async_copy(p3_hbm.at[1].at[pidx], rb, sem)
        pltpu.async_copy(g3_hbm.at[pidx], rc, sem)
        pltpu.async_copy(dinv_hbm.at[pidx], dv, sem)
        pltpu.make_async_copy(p3_hbm.at[0].at[pidx], ra, sem).wait()
        pltpu.make_async_copy(p3_hbm.at[1].at[pidx], rb, sem).wait()
        pltpu.make_async_copy(g3_hbm.at[pidx], rc, sem).wait()
        pltpu.make_async_copy(dinv_hbm.at[pidx], dv, sem).wait()

        def crow(t, carry):
            ev = dv[pl.ds(t * 16, 16)]
            for k in range(16):
                sv = jnp.full((16,), ev[k], jnp.float32)
                rr = t * 16 + k
                for j in range(D // 16):
                    cs = pl.ds(j * 16, 16)
                    ra[rr, cs] = (ra[rr, cs] + rb[rr, cs] + rc[rr, cs]) * sv \
                        + bv[cs]
            return carry

        lax.fori_loop(0, P // 16, crow, 0)
        pltpu.sync_copy(ra, out_hbm)


BR = 1000
G = N // BR


def _tc_mm_body(x_ref, w_ref, hw_ref):
    hw_ref[...] = jnp.dot(x_ref[...], w_ref[...],
                          preferred_element_type=jnp.float32)


def _tc_mm(x, w):
    return pl.pallas_call(
        _tc_mm_body,
        grid=(G,),
        in_specs=[
            pl.BlockSpec((BR, D), lambda i: (i, 0)),
            pl.BlockSpec((D, D), lambda i: (0, 0)),
        ],
        out_specs=pl.BlockSpec((BR, D), lambda i: (i, 0)),
        out_shape=jax.ShapeDtypeStruct((N, D), jnp.float32),
    )(x, w)


def _tc_pre_body(degp_ref, hw_ref, g_ref, dinv_ref):
    deg = degp_ref[0] + degp_ref[1] + 1.0
    dinv = lax.rsqrt(deg)
    g_ref[...] = dinv * hw_ref[...]
    dinv_ref[...] = dinv


def _tc_pre(degp, hw):
    return pl.pallas_call(
        _tc_pre_body,
        grid=(G,),
        in_specs=[
            pl.BlockSpec((NC, BR, 1), lambda i: (0, i, 0)),
            pl.BlockSpec((BR, D), lambda i: (i, 0)),
        ],
        out_specs=[
            pl.BlockSpec((BR, D), lambda i: (i, 0)),
            pl.BlockSpec((BR, 1), lambda i: (i, 0)),
        ],
        out_shape=[
            jax.ShapeDtypeStruct((N, D), jnp.float32),
            jax.ShapeDtypeStruct((N, 1), jnp.float32),
        ],
    )(degp, hw)


def _tc_mid_body(parts_ref, g_ref, dinv_ref, b_ref, w_ref, gn_ref):
    s = parts_ref[0] + parts_ref[1] + g_ref[...]
    emb = dinv_ref[...] * s + b_ref[...]
    h = jnp.tanh(emb)
    hw = jnp.dot(h, w_ref[...], preferred_element_type=jnp.float32)
    gn_ref[...] = dinv_ref[...] * hw


def _tc_mid(parts, g, dinv, b, w):
    return pl.pallas_call(
        _tc_mid_body,
        grid=(G,),
        in_specs=[
            pl.BlockSpec((NC, BR, D), lambda i: (0, i, 0)),
            pl.BlockSpec((BR, D), lambda i: (i, 0)),
            pl.BlockSpec((BR, 1), lambda i: (i, 0)),
            pl.BlockSpec((1, D), lambda i: (0, 0)),
            pl.BlockSpec((D, D), lambda i: (0, 0)),
        ],
        out_specs=pl.BlockSpec((BR, D), lambda i: (i, 0)),
        out_shape=jax.ShapeDtypeStruct((N, D), jnp.float32),
    )(parts, g, dinv, b, w)


def kernel(x, edge_index, edge_weight, pos, W1, b1, W2, b2, W3, b3):
    src = edge_index[0]
    dst = edge_index[1]

    pad = EPAD - E
    fillr = (jnp.arange(pad, dtype=jnp.int32) * 131) % N
    psrc = jnp.concatenate([src, fillr]).reshape(NW, SCN, SUBS, K)
    pdst = jnp.concatenate([dst, fillr]).reshape(NW, SCN, SUBS, K)
    pew = jnp.concatenate(
        [edge_weight, jnp.zeros((pad,), jnp.float32)]).reshape(
            NW, SCN, SUBS, K)

    hw1 = _tc_mm(x, W1)
    degp = _sc_deg(pdst, pew)
    degp3 = degp.reshape(NC, NPAD, 1)

    g1, dinv = _tc_pre(degp3, hw1)

    p1 = _sc_agg(g1, psrc, pdst, pew)
    g2 = _tc_mid(p1, g1, dinv, b1.reshape(1, D), W2)
    p2 = _sc_agg(g2, psrc, pdst, pew)
    g3 = _tc_mid(p2, g2, dinv, b2.reshape(1, D), W3)
    p3 = _sc_agg(g3, psrc, pdst, pew)

    rows = _sc_tail(p3, g3, dinv.reshape(N), b3, pos)
    fill = jnp.full((P, D), -float(DEPTH), dtype=jnp.float32)
    embs = jnp.where((pos == -1)[:, None], fill, rows)
    return embs[None]
